# baseline (speedup 1.0000x reference)
import jax
import jax.numpy as jnp
from jax.experimental import pallas as pl
from jax.experimental.pallas import tpu as pltpu

BSZ, L, D = 4, 512, 128
HEADS, OUT_CH = 2, 64
NB = 4


def _gat_kernel(x_ref, graph_ref, w_ref, att_src_ref, att_dst_ref, bias_ref,
                out_ref, maskf_ref, maskb_ref):
    @pl.when(pl.program_id(0) == 0)
    def _build_mask():
        ii = jax.lax.broadcasted_iota(jnp.int32, (L, L), 0)
        jj = jax.lax.broadcasted_iota(jnp.int32, (L, L), 1)
        m = ((graph_ref[...].T != 0.0) | (ii == jj)).astype(jnp.float32)
        maskf_ref[...] = m
        maskb_ref[...] = m.astype(jnp.bfloat16)

    maskf = maskf_ref[...]
    maskb = maskb_ref[...]
    ones_row = jnp.ones((1, L), dtype=jnp.float32)
    bias = bias_ref[...]
    for nb in range(NB):
        _gat_block(x_ref, w_ref, att_src_ref, att_dst_ref, out_ref,
                   maskf, maskb, ones_row, bias, nb)


def _gat_block(x_ref, w_ref, att_src_ref, att_dst_ref, out_ref,
               maskf, maskb, ones_row, bias, nb):
    x = x_ref[nb]
    h = jax.lax.dot_general(x, w_ref[...], (((1,), (1,)), ((), ())),
                            preferred_element_type=jnp.float32)
    z64 = jnp.zeros((OUT_CH, 4), jnp.float32)
    blk = []
    for hd in range(HEADS):
        aT = att_src_ref[hd:hd + 1, :].T
        dT = att_dst_ref[hd:hd + 1, :].T
        quad = jnp.concatenate([aT, 0.2 * aT, dT, 0.2 * dT], axis=1)
        blk.append(jnp.concatenate(
            [z64] * hd + [quad] + [z64] * (HEADS - 1 - hd), axis=1))
    m8 = jnp.concatenate(blk, axis=0)
    sd8 = jax.lax.dot_general(h, m8, (((1,), (0,)), ((), ())),
                              preferred_element_type=jnp.float32)
    ex8 = jnp.exp(sd8)
    ones_rowb = jnp.ones((1, L), dtype=jnp.bfloat16)
    outs = []
    for hd in range(HEADS):
        hh = h[:, hd * OUT_CH:(hd + 1) * OUT_CH]
        asr = att_src_ref[hd:hd + 1, :]
        k = 4 * hd
        s_row = jax.lax.dot_general(asr, hh, (((1,), (1,)), ((), ())),
                                    preferred_element_type=jnp.float32)
        d_colb = sd8[:, k + 2:k + 3].astype(jnp.bfloat16)
        d_bc = jax.lax.dot_general(d_colb, ones_rowb, (((1,), (0,)), ((), ())),
                                   preferred_element_type=jnp.float32)
        g = d_bc >= -s_row
        E1 = jnp.where(g, maskf, 0.0).astype(jnp.bfloat16)
        E2 = maskb - E1
        R1 = jnp.concatenate(
            [(hh * ex8[:, k:k + 1]).astype(jnp.bfloat16),
             ex8[:, k:k + 1].astype(jnp.bfloat16)], axis=1)
        R2 = jnp.concatenate(
            [(hh * ex8[:, k + 1:k + 2]).astype(jnp.bfloat16),
             ex8[:, k + 1:k + 2].astype(jnp.bfloat16)], axis=1)
        num1 = jax.lax.dot_general(E1, R1, (((1,), (0,)), ((), ())),
                                   preferred_element_type=jnp.float32)
        num2 = jax.lax.dot_general(E2, R2, (((1,), (0,)), ((), ())),
                                   preferred_element_type=jnp.float32)
        tot = num1 * ex8[:, k + 2:k + 3] + num2 * ex8[:, k + 3:k + 4]
        outs.append(tot[:, :OUT_CH] / (tot[:, OUT_CH:] + 1e-16))
    out = jnp.concatenate(outs, axis=1) + bias
    out_ref[nb] = jnp.maximum(out, 0.0) + x


@jax.jit
def _gat(x, graph, W, att_src, att_dst, bias):
    bias2 = bias.reshape(1, HEADS * OUT_CH)
    return pl.pallas_call(
        _gat_kernel,
        grid=(BSZ // NB,),
        in_specs=[
            pl.BlockSpec((NB, L, D), lambda b: (b, 0, 0)),
            pl.BlockSpec((L, L), lambda b: (0, 0)),
            pl.BlockSpec((HEADS * OUT_CH, D), lambda b: (0, 0)),
            pl.BlockSpec((HEADS, OUT_CH), lambda b: (0, 0)),
            pl.BlockSpec((HEADS, OUT_CH), lambda b: (0, 0)),
            pl.BlockSpec((1, HEADS * OUT_CH), lambda b: (0, 0)),
        ],
        out_specs=pl.BlockSpec((NB, L, D), lambda b: (b, 0, 0)),
        out_shape=jax.ShapeDtypeStruct((BSZ, L, HEADS * OUT_CH), jnp.float32),
        scratch_shapes=[pltpu.VMEM((L, L), jnp.float32),
                        pltpu.VMEM((L, L), jnp.bfloat16)],
    )(x, graph, W, att_src, att_dst, bias2)


def kernel(x, graph, W, att_src, att_dst, bias):
    return _gat(x, graph, W, att_src, att_dst, bias)


# speedup vs baseline: 1.1135x; 1.1135x over previous
import jax
import jax.numpy as jnp
from jax.experimental import pallas as pl
from jax.experimental.pallas import tpu as pltpu

BSZ, L, D = 4, 512, 128
HEADS, OUT_CH = 2, 64
NB = 1


def _gat_kernel(x_ref, graph_ref, w_ref, att_src_ref, att_dst_ref, bias_ref,
                out_ref, maskf_ref, maskb_ref):
    @pl.when(pl.program_id(0) == 0)
    def _build_mask():
        ii = jax.lax.broadcasted_iota(jnp.int32, (L, L), 0)
        jj = jax.lax.broadcasted_iota(jnp.int32, (L, L), 1)
        m = ((graph_ref[...].T != 0.0) | (ii == jj)).astype(jnp.float32)
        maskf_ref[...] = m
        maskb_ref[...] = m.astype(jnp.bfloat16)

    maskf = maskf_ref[...]
    maskb = maskb_ref[...]
    ones_row = jnp.ones((1, L), dtype=jnp.float32)
    bias = bias_ref[...]
    for nb in range(NB):
        _gat_block(x_ref, w_ref, att_src_ref, att_dst_ref, out_ref,
                   maskf, maskb, ones_row, bias, nb)


def _gat_block(x_ref, w_ref, att_src_ref, att_dst_ref, out_ref,
               maskf, maskb, ones_row, bias, nb):
    x = x_ref[nb]
    h = jax.lax.dot_general(x, w_ref[...], (((1,), (1,)), ((), ())),
                            preferred_element_type=jnp.float32)
    z64 = jnp.zeros((OUT_CH, 4), jnp.float32)
    blk = []
    for hd in range(HEADS):
        aT = att_src_ref[hd:hd + 1, :].T
        dT = att_dst_ref[hd:hd + 1, :].T
        quad = jnp.concatenate([aT, 0.2 * aT, dT, 0.2 * dT], axis=1)
        blk.append(jnp.concatenate(
            [z64] * hd + [quad] + [z64] * (HEADS - 1 - hd), axis=1))
    m8 = jnp.concatenate(blk, axis=0)
    sd8 = jax.lax.dot_general(h, m8, (((1,), (0,)), ((), ())),
                              preferred_element_type=jnp.float32)
    ex8 = jnp.exp(sd8)
    ones_rowb = jnp.ones((1, L), dtype=jnp.bfloat16)
    outs = []
    for hd in range(HEADS):
        hh = h[:, hd * OUT_CH:(hd + 1) * OUT_CH]
        asr = att_src_ref[hd:hd + 1, :]
        k = 4 * hd
        s_row = jax.lax.dot_general(asr, hh, (((1,), (1,)), ((), ())),
                                    preferred_element_type=jnp.float32)
        d_colb = sd8[:, k + 2:k + 3].astype(jnp.bfloat16)
        d_bc = jax.lax.dot_general(d_colb, ones_rowb, (((1,), (0,)), ((), ())),
                                   preferred_element_type=jnp.float32)
        g = d_bc >= -s_row
        E1 = jnp.where(g, maskf, 0.0).astype(jnp.bfloat16)
        E2 = maskb - E1
        R1 = jnp.concatenate(
            [(hh * ex8[:, k:k + 1]).astype(jnp.bfloat16),
             ex8[:, k:k + 1].astype(jnp.bfloat16)], axis=1)
        R2 = jnp.concatenate(
            [(hh * ex8[:, k + 1:k + 2]).astype(jnp.bfloat16),
             ex8[:, k + 1:k + 2].astype(jnp.bfloat16)], axis=1)
        num1 = jax.lax.dot_general(E1, R1, (((1,), (0,)), ((), ())),
                                   preferred_element_type=jnp.float32)
        num2 = jax.lax.dot_general(E2, R2, (((1,), (0,)), ((), ())),
                                   preferred_element_type=jnp.float32)
        tot = num1 * ex8[:, k + 2:k + 3] + num2 * ex8[:, k + 3:k + 4]
        outs.append(tot[:, :OUT_CH] / (tot[:, OUT_CH:] + 1e-16))
    out = jnp.concatenate(outs, axis=1) + bias
    out_ref[nb] = jnp.maximum(out, 0.0) + x


@jax.jit
def _gat(x, graph, W, att_src, att_dst, bias):
    bias2 = bias.reshape(1, HEADS * OUT_CH)
    return pl.pallas_call(
        _gat_kernel,
        grid=(BSZ // NB,),
        in_specs=[
            pl.BlockSpec((NB, L, D), lambda b: (b, 0, 0)),
            pl.BlockSpec((L, L), lambda b: (0, 0)),
            pl.BlockSpec((HEADS * OUT_CH, D), lambda b: (0, 0)),
            pl.BlockSpec((HEADS, OUT_CH), lambda b: (0, 0)),
            pl.BlockSpec((HEADS, OUT_CH), lambda b: (0, 0)),
            pl.BlockSpec((1, HEADS * OUT_CH), lambda b: (0, 0)),
        ],
        out_specs=pl.BlockSpec((NB, L, D), lambda b: (b, 0, 0)),
        out_shape=jax.ShapeDtypeStruct((BSZ, L, HEADS * OUT_CH), jnp.float32),
        scratch_shapes=[pltpu.VMEM((L, L), jnp.float32),
                        pltpu.VMEM((L, L), jnp.bfloat16)],
    )(x, graph, W, att_src, att_dst, bias2)


def kernel(x, graph, W, att_src, att_dst, bias):
    return _gat(x, graph, W, att_src, att_dst, bias)
